# Initial kernel scaffold; baseline (speedup 1.0000x reference)
import functools

import jax
import jax.numpy as jnp
from jax import lax
from jax.experimental import pallas as pl
from jax.experimental.pallas import tpu as pltpu
from jax.experimental.pallas import tpu_sc as plsc

N = 10000
H = 8
E = 320000
ETOT = E + N
NW = 32
B = 128
NB = -(-ETOT // (NW * B))
EPAD = NW * B * NB
ROWS_PER_TILE = -(-(N + 1) // 16)
ACC_ROWS = 16 * ROWS_PER_TILE
BN = 400


def _proj_body(x_ref, w_ref, o_ref):
    o_ref[...] = jnp.dot(x_ref[...], w_ref[...], preferred_element_type=jnp.float32)


def _project(x, wt):
    k, r = wt.shape
    return pl.pallas_call(
        _proj_body,
        grid=(N // BN,),
        in_specs=[pl.BlockSpec((BN, k), lambda i: (i, 0)),
                  pl.BlockSpec((k, r), lambda i: (0, 0))],
        out_specs=pl.BlockSpec((BN, r), lambda i: (i, 0)),
        out_shape=jax.ShapeDtypeStruct((N, r), jnp.float32),
    )(x, wt)


def _mid_body(acc_ref, b_ref, exp8_ref, w_ref, o_ref):
    a = acc_ref[...]
    num = a[0][:, :64] + a[1][:, :64]
    den = a[0][:, 64:72] + a[1][:, 64:72]
    den_e = jnp.dot(den, exp8_ref[...], preferred_element_type=jnp.float32) + 1e-16
    hb = num / den_e + b_ref[...]
    hb = jnp.where(hb > 0, hb, jnp.exp(hb) - 1.0)
    o_ref[...] = jnp.dot(hb, w_ref[...], preferred_element_type=jnp.float32)


def _post_body(acc_ref, b_ref, exp16_ref, mean_ref, o_ref):
    a = acc_ref[...]
    num = a[0][:, :128] + a[1][:, :128]
    den = a[0][:, 128:136] + a[1][:, 128:136]
    den_e = jnp.dot(den, exp16_ref[...], preferred_element_type=jnp.float32) + 1e-16
    ratio = num / den_e
    z = jnp.dot(ratio, mean_ref[...], preferred_element_type=jnp.float32) + b_ref[...]
    m = jnp.max(z, axis=1, keepdims=True)
    s = jnp.sum(jnp.exp(z - m), axis=1, keepdims=True)
    o_ref[...] = z - m - jnp.log(s)


def _make_edge_kernel(C):
    D = H * C
    R = D + 16
    mesh = plsc.VectorSubcoreMesh(core_axis_name="c", subcore_axis_name="s")

    @functools.partial(
        pl.kernel,
        out_type=jax.ShapeDtypeStruct((2, ACC_ROWS, R), jnp.float32),
        mesh=mesh,
        scratch_types=[
            pltpu.VMEM((NB, B), jnp.int32),
            pltpu.VMEM((NB, B), jnp.int32),
            pltpu.VMEM((B, R), jnp.float32),
            pltpu.VMEM((B, R), jnp.float32),
            pltpu.VMEM((B, R), jnp.float32),
            pltpu.VMEM((16,), jnp.float32),
            pltpu.VMEM_SHARED((ACC_ROWS, R), jnp.float32),
            pltpu.SemaphoreType.DMA,
            pltpu.SemaphoreType.DMA,
        ],
    )
    def edge_kernel(table_hbm, src_hbm, dst_hbm, m_hbm, zeros_hbm, out_hbm,
                    src_v, dst_v, rows_j, rows_i, prod, m_v, acc, sem0, sem1):
        cid = lax.axis_index("c")
        sid = lax.axis_index("s")
        wid = cid * 16 + sid
        pltpu.sync_copy(zeros_hbm, acc.at[pl.ds(sid * ROWS_PER_TILE, ROWS_PER_TILE)])
        pltpu.sync_copy(zeros_hbm.at[pl.ds(0, B)], prod)
        pltpu.sync_copy(src_hbm.at[wid], src_v)
        pltpu.sync_copy(dst_hbm.at[wid], dst_v)
        pltpu.sync_copy(m_hbm, m_v)
        lane = lax.iota(jnp.int32, 16)
        m_vec = m_v[...]
        m_sc = [jnp.max(jnp.where(lane == h, m_vec, -3e38)) for h in range(H)]
        plsc.subcore_barrier()

        hpb = 32 // C

        def batch(j, carry):
            cp0 = pltpu.async_copy(table_hbm.at[src_v.at[j]], rows_j, sem0)
            cp1 = pltpu.async_copy(table_hbm.at[dst_v.at[j]], rows_i, sem1)
            cp0.wait()
            cp1.wait()

            def group(g, c2):
                rowb = g * 16 + lane

                def colj(c):
                    return plsc.load_gather(rows_j, [rowb, jnp.full((16,), c, jnp.int32)])

                def coli(c):
                    return plsc.load_gather(rows_i, [rowb, jnp.full((16,), c, jnp.int32)])

                for blk in range(D // 32):
                    xj = [colj(32 * blk + k) for k in range(32)]
                    for hh in range(hpb):
                        h = blk * hpb + hh
                        lg = None
                        for cc in range(C):
                            t = xj[hh * C + cc] * coli(32 * blk + hh * C + cc)
                            lg = t if lg is None else lg + t
                        base = colj(D + h) + coli(D + H + h)
                        sig = 1.0 / (1.0 + jnp.exp(-lg))
                        a = base * sig
                        a = jnp.where(a >= 0.0, a, 0.2 * a)
                        ex = jnp.exp(a - m_sc[h])
                        for cc in range(C):
                            c = 32 * blk + hh * C + cc
                            plsc.store_scatter(
                                prod, [rowb, jnp.full((16,), c, jnp.int32)],
                                xj[hh * C + cc] * ex)
                        plsc.store_scatter(
                            prod, [rowb, jnp.full((16,), D + h, jnp.int32)], ex)
                return c2

            lax.fori_loop(0, B // 16, group, 0)
            pltpu.sync_copy(prod, acc.at[dst_v.at[j]], add=True)
            return carry

        lax.fori_loop(0, NB, batch, 0)
        plsc.subcore_barrier()
        pltpu.sync_copy(acc.at[pl.ds(sid * ROWS_PER_TILE, ROWS_PER_TILE)],
                        out_hbm.at[cid, pl.ds(sid * ROWS_PER_TILE, ROWS_PER_TILE)])

    return edge_kernel


_edge_kernel_1 = _make_edge_kernel(8)
_edge_kernel_2 = _make_edge_kernel(16)


def _ext_weights(W, att_l, att_r, C):
    D = H * C
    al = att_l.reshape(H, C)
    ar = att_r.reshape(H, C)
    eye = jnp.eye(H, dtype=jnp.float32)
    A_L = (al[:, :, None] * eye[:, None, :]).reshape(D, H)
    A_R = (ar[:, :, None] * eye[:, None, :]).reshape(D, H)
    WT = W.T
    return jnp.concatenate([WT, WT @ A_L, WT @ A_R], axis=1)


def kernel(x, edge_index, W1, att_l1, att_r1, b1, W2, att_l2, att_r2, b2):
    src, dst = edge_index[0], edge_index[1]
    loop = jnp.arange(N, dtype=src.dtype)
    valid = jnp.concatenate([src != dst, jnp.ones((N,), bool)])
    src_all = jnp.concatenate([src, loop])
    dst_all = jnp.where(valid, jnp.concatenate([dst, loop]), N)
    pad = EPAD - ETOT
    src_p = jnp.concatenate([src_all, jnp.zeros((pad,), src.dtype)]).reshape(NW, NB, B)
    dst_p = jnp.concatenate([dst_all, jnp.full((pad,), N, src.dtype)]).reshape(NW, NB, B)

    wt1 = _ext_weights(W1, att_l1, att_r1, 8)
    wt2 = _ext_weights(W2, att_l2, att_r2, 16)

    table1 = _project(x, wt1)
    table1f = jnp.concatenate([table1, table1[:1]], axis=0)
    m1 = jnp.maximum(jnp.max(table1[:, 64:72], 0) + jnp.max(table1[:, 72:80], 0), 0.0)
    m1 = jnp.pad(m1, (0, 8))
    zeros1 = jnp.zeros((ROWS_PER_TILE, 80), jnp.float32)
    acc1 = _edge_kernel_1(table1f, src_p, dst_p, m1, zeros1)

    exp8 = jnp.kron(jnp.eye(8), jnp.ones((1, 8))).astype(jnp.float32)
    table2 = pl.pallas_call(
        _mid_body,
        grid=(N // BN,),
        in_specs=[pl.BlockSpec((2, BN, 80), lambda i: (0, i, 0)),
                  pl.BlockSpec((1, 64), lambda i: (0, 0)),
                  pl.BlockSpec((8, 64), lambda i: (0, 0)),
                  pl.BlockSpec((128, 144), lambda i: (0, 0))],
        out_specs=pl.BlockSpec((BN, 144), lambda i: (i, 0)),
        out_shape=jax.ShapeDtypeStruct((N, 144), jnp.float32),
    )(acc1, b1.reshape(1, 64), exp8, wt2)

    table2f = jnp.concatenate([table2, table2[:1]], axis=0)
    m2 = jnp.maximum(jnp.max(table2[:, 128:136], 0) + jnp.max(table2[:, 136:144], 0), 0.0)
    m2 = jnp.pad(m2, (0, 8))
    zeros2 = jnp.zeros((ROWS_PER_TILE, 144), jnp.float32)
    acc2 = _edge_kernel_2(table2f, src_p, dst_p, m2, zeros2)

    exp16 = jnp.kron(jnp.eye(8), jnp.ones((1, 16))).astype(jnp.float32)
    meanm = (jnp.kron(jnp.ones((8, 1)), jnp.eye(16)) / 8.0).astype(jnp.float32)
    logp = pl.pallas_call(
        _post_body,
        grid=(N // BN,),
        in_specs=[pl.BlockSpec((2, BN, 144), lambda i: (0, i, 0)),
                  pl.BlockSpec((1, 16), lambda i: (0, 0)),
                  pl.BlockSpec((8, 128), lambda i: (0, 0)),
                  pl.BlockSpec((128, 16), lambda i: (0, 0))],
        out_specs=pl.BlockSpec((BN, 16), lambda i: (i, 0)),
        out_shape=jax.ShapeDtypeStruct((N, 16), jnp.float32),
    )(acc2, b2.reshape(1, 16), exp16, meanm)

    return logp, jnp.zeros((), jnp.float32)


# speedup vs baseline: 39.0561x; 39.0561x over previous
import functools

import jax
import jax.numpy as jnp
from jax import lax
from jax.experimental import pallas as pl
from jax.experimental.pallas import tpu as pltpu
from jax.experimental.pallas import tpu_sc as plsc

N = 10000
H = 8
E = 320000
ETOT = E + N
NW = 32
B = 128
NB = -(-ETOT // (NW * B))
EPAD = NW * B * NB
ROWS_PER_TILE = 8 * (-(-(N + 1) // (16 * 8)))
ACC_ROWS = 16 * ROWS_PER_TILE
R = 128
BN = 400
BE = 1024


def _proj_body(x_ref, w_ref, o_ref):
    o_ref[...] = jnp.dot(x_ref[...], w_ref[...], preferred_element_type=jnp.float32)


def _project(x, wt):
    k, r = wt.shape
    return pl.pallas_call(
        _proj_body,
        grid=(N // BN,),
        in_specs=[pl.BlockSpec((BN, k), lambda i: (i, 0)),
                  pl.BlockSpec((k, r), lambda i: (0, 0))],
        out_specs=pl.BlockSpec((BN, r), lambda i: (i, 0)),
        out_shape=jax.ShapeDtypeStruct((N, r), jnp.float32),
    )(x, wt)


def _edge1_body(xj_ref, xi_ref, m_ref, s_ref, rx_ref, o_ref):
    xj = xj_ref[...]
    xi = xi_ref[...]
    xjf = xj[:, :64]
    t = xjf * xi[:, :64]
    lg = jnp.dot(t, s_ref[...], preferred_element_type=jnp.float32)
    base = xj[:, 64:72] + xi[:, 72:80]
    sig = 1.0 / (1.0 + jnp.exp(-lg))
    a = base * sig
    a = jnp.where(a >= 0.0, a, 0.2 * a)
    ex = jnp.exp(a - m_ref[...])
    exr = jnp.dot(ex, rx_ref[...], preferred_element_type=jnp.float32)
    o_ref[...] = jnp.concatenate(
        [xjf * exr, ex, jnp.zeros((xj.shape[0], 56), jnp.float32)], axis=1)


def _edge2_body(xj_ref, xi_ref, m_ref, s_ref, al_ref, ar_ref, rx_ref,
                oa_ref, ob_ref):
    xj = xj_ref[...]
    xi = xi_ref[...]
    t = xj * xi
    lg = jnp.dot(t, s_ref[...], preferred_element_type=jnp.float32)
    base = (jnp.dot(xj, al_ref[...], preferred_element_type=jnp.float32)
            + jnp.dot(xi, ar_ref[...], preferred_element_type=jnp.float32))
    sig = 1.0 / (1.0 + jnp.exp(-lg))
    a = base * sig
    a = jnp.where(a >= 0.0, a, 0.2 * a)
    ex = jnp.exp(a - m_ref[...])
    z = jnp.zeros((xj.shape[0], 60), jnp.float32)
    exa = ex[:, :4]
    exb = ex[:, 4:]
    rxm = rx_ref[...]
    oa_ref[...] = jnp.concatenate(
        [xj[:, :64] * jnp.dot(exa, rxm, preferred_element_type=jnp.float32),
         exa, z], axis=1)
    ob_ref[...] = jnp.concatenate(
        [xj[:, 64:] * jnp.dot(exb, rxm, preferred_element_type=jnp.float32),
         exb, z], axis=1)


def _mid_body(acc_ref, b_ref, exp8_ref, w_ref, o1_ref, o2_ref):
    a = acc_ref[...]
    num = a[0][:, :64] + a[1][:, :64]
    den = a[0][:, 64:72] + a[1][:, 64:72]
    den_e = jnp.dot(den, exp8_ref[...], preferred_element_type=jnp.float32) + 1e-16
    hb = num / den_e + b_ref[...]
    hb = jnp.where(hb > 0, hb, jnp.exp(hb) - 1.0)
    z = jnp.dot(hb, w_ref[...], preferred_element_type=jnp.float32)
    o1_ref[...] = z[:, :128]
    o2_ref[...] = z[:, 128:]


def _post_body(aa_ref, ab_ref, b_ref, exp4_ref, mean_ref, o_ref):
    aa = aa_ref[...]
    ab = ab_ref[...]
    num_a = aa[0][:, :64] + aa[1][:, :64]
    den_a = aa[0][:, 64:68] + aa[1][:, 64:68]
    num_b = ab[0][:, :64] + ab[1][:, :64]
    den_b = ab[0][:, 64:68] + ab[1][:, 64:68]
    e4 = exp4_ref[...]
    r_a = num_a / (jnp.dot(den_a, e4, preferred_element_type=jnp.float32) + 1e-16)
    r_b = num_b / (jnp.dot(den_b, e4, preferred_element_type=jnp.float32) + 1e-16)
    mn = mean_ref[...]
    z = (jnp.dot(r_a, mn, preferred_element_type=jnp.float32)
         + jnp.dot(r_b, mn, preferred_element_type=jnp.float32) + b_ref[...])
    m = jnp.max(z, axis=1, keepdims=True)
    s = jnp.sum(jnp.exp(z - m), axis=1, keepdims=True)
    o_ref[...] = z - m - jnp.log(s)


@functools.lru_cache(maxsize=None)
def _make_gather_kernel():
    mesh = plsc.VectorSubcoreMesh(core_axis_name="c", subcore_axis_name="s")

    @functools.partial(
        pl.kernel,
        out_type=(jax.ShapeDtypeStruct((EPAD, R), jnp.float32),
                  jax.ShapeDtypeStruct((EPAD, R), jnp.float32)),
        mesh=mesh,
        scratch_types=[
            pltpu.VMEM((NB, B), jnp.int32),
            pltpu.VMEM((NB, B), jnp.int32),
            pltpu.VMEM((B, R), jnp.float32),
            pltpu.VMEM((B, R), jnp.float32),
            pltpu.SemaphoreType.DMA,
            pltpu.SemaphoreType.DMA,
        ],
    )
    def gather_kernel(table_hbm, src_hbm, dst_hbm, oj_hbm, oi_hbm,
                      src_v, dst_v, rows_j, rows_i, sem0, sem1):
        cid = lax.axis_index("c")
        sid = lax.axis_index("s")
        wid = cid * 16 + sid
        pltpu.sync_copy(src_hbm.at[wid], src_v)
        pltpu.sync_copy(dst_hbm.at[wid], dst_v)

        def batch(j, carry):
            cp0 = pltpu.async_copy(table_hbm.at[src_v.at[j]], rows_j, sem0)
            cp1 = pltpu.async_copy(table_hbm.at[dst_v.at[j]], rows_i, sem1)
            cp0.wait()
            cp1.wait()
            base = (wid * NB + j) * B
            pltpu.sync_copy(rows_j, oj_hbm.at[pl.ds(base, B)])
            pltpu.sync_copy(rows_i, oi_hbm.at[pl.ds(base, B)])
            return carry

        lax.fori_loop(0, NB, batch, 0)

    return gather_kernel


@functools.lru_cache(maxsize=None)
def _make_scatter_kernel():
    mesh = plsc.VectorSubcoreMesh(core_axis_name="c", subcore_axis_name="s")

    @functools.partial(
        pl.kernel,
        out_type=jax.ShapeDtypeStruct((2, ACC_ROWS, R), jnp.float32),
        mesh=mesh,
        scratch_types=[
            pltpu.VMEM((NB, B), jnp.int32),
            pltpu.VMEM((B, R), jnp.float32),
            pltpu.VMEM_SHARED((ACC_ROWS, R), jnp.float32),
        ],
    )
    def scatter_kernel(prod_hbm, dst_hbm, zeros_hbm, out_hbm,
                       dst_v, prod_v, acc):
        cid = lax.axis_index("c")
        sid = lax.axis_index("s")
        wid = cid * 16 + sid
        slab = pl.ds(sid * ROWS_PER_TILE, ROWS_PER_TILE)
        pltpu.sync_copy(zeros_hbm, acc.at[slab])
        pltpu.sync_copy(dst_hbm.at[wid], dst_v)
        plsc.subcore_barrier()

        def batch(j, carry):
            base = (wid * NB + j) * B
            pltpu.sync_copy(prod_hbm.at[pl.ds(base, B)], prod_v)
            pltpu.sync_copy(prod_v, acc.at[dst_v.at[j]], add=True)
            return carry

        lax.fori_loop(0, NB, batch, 0)
        plsc.subcore_barrier()
        pltpu.sync_copy(acc.at[slab], out_hbm.at[cid, slab])

    return scatter_kernel


def _att_block(att, C):
    a = att.reshape(H, C)
    eye = jnp.eye(H, dtype=jnp.float32)
    return (a[:, :, None] * eye[:, None, :]).reshape(H * C, H)


def _scatter(prod, dst_p):
    zeros = jnp.zeros((ROWS_PER_TILE, R), jnp.float32)
    return _make_scatter_kernel()(prod, dst_p, zeros)


def kernel(x, edge_index, W1, att_l1, att_r1, b1, W2, att_l2, att_r2, b2):
    src, dst = edge_index[0], edge_index[1]
    loop = jnp.arange(N, dtype=src.dtype)
    valid = jnp.concatenate([src != dst, jnp.ones((N,), bool)])
    src_all = jnp.concatenate([src, loop])
    dst_all = jnp.concatenate([dst, loop])
    dst_g = jnp.where(valid, dst_all, 0)
    dst_s = jnp.where(valid, dst_all, N)
    pad = EPAD - ETOT
    src_p = jnp.concatenate([src_all, jnp.zeros((pad,), src.dtype)]).reshape(NW, NB, B)
    dstg_p = jnp.concatenate([dst_g, jnp.zeros((pad,), src.dtype)]).reshape(NW, NB, B)
    dsts_p = jnp.concatenate([dst_s, jnp.full((pad,), N, src.dtype)]).reshape(NW, NB, B)

    W1T = W1.T
    wt1 = jnp.concatenate(
        [W1T, W1T @ _att_block(att_l1, 8), W1T @ _att_block(att_r1, 8),
         jnp.zeros((128, 48), jnp.float32)], axis=1)
    W2T = W2.T
    wt2 = jnp.concatenate(
        [W2T, W2T @ _att_block(att_l2, 16), W2T @ _att_block(att_r2, 16)],
        axis=1)
    al2 = _att_block(att_l2, 16)
    ar2 = _att_block(att_r2, 16)

    table1 = _project(x, wt1)
    m1 = jnp.maximum(jnp.max(table1[:, 64:72], 0) + jnp.max(table1[:, 72:80], 0), 0.0)
    rows_j1, rows_i1 = _make_gather_kernel()(table1, src_p, dstg_p)
    seg1 = jnp.kron(jnp.eye(8), jnp.ones((8, 1))).astype(jnp.float32)
    rx1 = jnp.kron(jnp.eye(8), jnp.ones((1, 8))).astype(jnp.float32)
    prod1 = pl.pallas_call(
        _edge1_body,
        grid=(EPAD // BE,),
        in_specs=[pl.BlockSpec((BE, R), lambda i: (i, 0)),
                  pl.BlockSpec((BE, R), lambda i: (i, 0)),
                  pl.BlockSpec((1, 8), lambda i: (0, 0)),
                  pl.BlockSpec((64, 8), lambda i: (0, 0)),
                  pl.BlockSpec((8, 64), lambda i: (0, 0))],
        out_specs=pl.BlockSpec((BE, R), lambda i: (i, 0)),
        out_shape=jax.ShapeDtypeStruct((EPAD, R), jnp.float32),
    )(rows_j1, rows_i1, m1.reshape(1, 8), seg1, rx1)
    acc1 = _scatter(prod1, dsts_p)

    exp8 = jnp.kron(jnp.eye(8), jnp.ones((1, 8))).astype(jnp.float32)
    table2, att2 = pl.pallas_call(
        _mid_body,
        grid=(N // BN,),
        in_specs=[pl.BlockSpec((2, BN, R), lambda i: (0, i, 0)),
                  pl.BlockSpec((1, 64), lambda i: (0, 0)),
                  pl.BlockSpec((8, 64), lambda i: (0, 0)),
                  pl.BlockSpec((64, 144), lambda i: (0, 0))],
        out_specs=[pl.BlockSpec((BN, 128), lambda i: (i, 0)),
                   pl.BlockSpec((BN, 16), lambda i: (i, 0))],
        out_shape=[jax.ShapeDtypeStruct((N, 128), jnp.float32),
                   jax.ShapeDtypeStruct((N, 16), jnp.float32)],
    )(acc1, b1.reshape(1, 64), exp8, wt2)

    m2 = jnp.maximum(jnp.max(att2[:, :8], 0) + jnp.max(att2[:, 8:], 0), 0.0)
    rows_j2, rows_i2 = _make_gather_kernel()(table2, src_p, dstg_p)
    seg2 = jnp.kron(jnp.eye(8), jnp.ones((16, 1))).astype(jnp.float32)
    rx2 = jnp.kron(jnp.eye(4), jnp.ones((1, 16))).astype(jnp.float32)
    prod2a, prod2b = pl.pallas_call(
        _edge2_body,
        grid=(EPAD // BE,),
        in_specs=[pl.BlockSpec((BE, R), lambda i: (i, 0)),
                  pl.BlockSpec((BE, R), lambda i: (i, 0)),
                  pl.BlockSpec((1, 8), lambda i: (0, 0)),
                  pl.BlockSpec((128, 8), lambda i: (0, 0)),
                  pl.BlockSpec((128, 8), lambda i: (0, 0)),
                  pl.BlockSpec((128, 8), lambda i: (0, 0)),
                  pl.BlockSpec((4, 64), lambda i: (0, 0))],
        out_specs=[pl.BlockSpec((BE, R), lambda i: (i, 0)),
                   pl.BlockSpec((BE, R), lambda i: (i, 0))],
        out_shape=[jax.ShapeDtypeStruct((EPAD, R), jnp.float32),
                   jax.ShapeDtypeStruct((EPAD, R), jnp.float32)],
    )(rows_j2, rows_i2, m2.reshape(1, 8), seg2, al2, ar2, rx2)
    acc2a = _scatter(prod2a, dsts_p)
    acc2b = _scatter(prod2b, dsts_p)

    exp4 = jnp.kron(jnp.eye(4), jnp.ones((1, 16))).astype(jnp.float32)
    meanm = (jnp.kron(jnp.ones((4, 1)), jnp.eye(16)) / 8.0).astype(jnp.float32)
    logp = pl.pallas_call(
        _post_body,
        grid=(N // BN,),
        in_specs=[pl.BlockSpec((2, BN, R), lambda i: (0, i, 0)),
                  pl.BlockSpec((2, BN, R), lambda i: (0, i, 0)),
                  pl.BlockSpec((1, 16), lambda i: (0, 0)),
                  pl.BlockSpec((4, 64), lambda i: (0, 0)),
                  pl.BlockSpec((64, 16), lambda i: (0, 0))],
        out_specs=pl.BlockSpec((BN, 16), lambda i: (i, 0)),
        out_shape=jax.ShapeDtypeStruct((N, 16), jnp.float32),
    )(acc2a, acc2b, b2.reshape(1, 16), exp4, meanm)

    return logp, jnp.zeros((), jnp.float32)
